# baseline (speedup 1.0000x reference)
import functools

import jax
import jax.numpy as jnp
from jax import lax
from jax.experimental import pallas as pl
from jax.experimental.pallas import tpu as pltpu
from jax.experimental.pallas import tpu_sc as plsc

_SC_PARAMS = pltpu.CompilerParams(needs_layout_passes=False,
                                  use_tc_tiling_on_sc=False)

N = 10000
E = 320000
D = 128
H = 128
HALF = 64
L = 8
P = 1000

NC = 2
NS = 16
NW = NC * NS
CH = E // NW
NSTEP = CH // 16
BLK = 128
NPAD = 10240
RPT = NPAD // NS
DUMP = N


def _leaky(x):
    return jnp.where(x > 0, x, 0.1 * x)


HA = 5008
HB = CH - HA


def _seg_body(src_hbm, dst_hbm, g_hbm, lo_hbm, hi_hbm, zinit_hbm, zout_hbm,
              src_v, dst_v, lo_v, hi_v, cidxa_v, cdfa_v, cidxb_v, cdfb_v,
              cd2_v, rows_v, zsp, gsem):
    core = lax.axis_index("c")
    sub = lax.axis_index("s")
    wid = sub * NC + core

    d2 = pltpu.async_copy(src_hbm.at[pl.ds(wid * CH, CH)], src_v, gsem)
    d3 = pltpu.async_copy(dst_hbm.at[pl.ds(wid * CH, CH)], dst_v, gsem)
    d4 = pltpu.async_copy(lo_hbm, lo_v, gsem)
    d5 = pltpu.async_copy(hi_hbm, hi_v, gsem)
    d6 = pltpu.async_copy(zinit_hbm, zsp.at[pl.ds(sub * RPT, RPT)], gsem)
    d2.wait(); d3.wait(); d4.wait(); d5.wait(); d6.wait()
    plsc.subcore_barrier()

    lo = lo_v[...]
    hi = hi_v[...]

    def gate(i, c):
        c0, c1 = c
        da = dst_v[pl.ds(i * 16, 16)]
        sa = src_v[pl.ds(i * 16, 16)]
        ma = (da >= lo) & (da < hi)
        plsc.store_compressed(cidxa_v.at[pl.ds(c0, 16)], sa, mask=ma)
        plsc.store_compressed(cdfa_v.at[pl.ds(c0, 16)], da, mask=ma)
        db = dst_v[pl.ds(HA + i * 16, 16)]
        sb = src_v[pl.ds(HA + i * 16, 16)]
        mb = (db >= lo) & (db < hi)
        plsc.store_compressed(cidxb_v.at[pl.ds(c1, 16)], sb, mask=mb)
        plsc.store_compressed(cdfb_v.at[pl.ds(c1, 16)], db, mask=mb)
        return (c0 + jnp.sum(ma.astype(jnp.int32)),
                c1 + jnp.sum(mb.astype(jnp.int32)))

    cnt0, cnt1 = lax.fori_loop(0, HB // 16, gate,
                               (jnp.int32(0), jnp.int32(0)))
    ia = HB // 16
    da = dst_v[pl.ds(ia * 16, 16)]
    sa = src_v[pl.ds(ia * 16, 16)]
    ma = (da >= lo) & (da < hi)
    plsc.store_compressed(cidxa_v.at[pl.ds(cnt0, 16)], sa, mask=ma)
    plsc.store_compressed(cdfa_v.at[pl.ds(cnt0, 16)], da, mask=ma)
    cnt0 = cnt0 + jnp.sum(ma.astype(jnp.int32))

    for k in range(8):
        cidxa_v[pl.ds(cnt0 + 16 * k, 16)] = jnp.zeros((16,), jnp.int32)
        cdfa_v[pl.ds(cnt0 + 16 * k, 16)] = jnp.full((16,), DUMP, jnp.int32)
        cidxb_v[pl.ds(cnt1 + 16 * k, 16)] = jnp.zeros((16,), jnp.int32)
        cdfb_v[pl.ds(cnt1 + 16 * k, 16)] = jnp.full((16,), DUMP, jnp.int32)

    def run_half(cidx_v, cdf_v, cnt):
        nblk = (cnt + (BLK - 1)) // BLK

        def start_gather(j, p):
            pltpu.async_copy(g_hbm.at[cidx_v.at[pl.ds(j * BLK, BLK)]],
                             rows_v.at[p], gsem)

        @pl.when(nblk > 0)
        def _():
            start_gather(0, 0)

        def block(j, _):
            p = lax.rem(j, 2)
            pltpu.make_async_copy(
                g_hbm.at[cidx_v.at[pl.ds(j * BLK, BLK)]],
                rows_v.at[p], gsem).wait()

            @pl.when(j + 1 < nblk)
            def _():
                start_gather(j + 1, 1 - p)

            for k in range(BLK // 16):
                cd2_v[p, pl.ds(16 * k, 16)] = \
                    cdf_v[pl.ds(j * BLK + 16 * k, 16)]
            pltpu.sync_copy(rows_v.at[p], zsp.at[cd2_v.at[p]], add=True)
            return 0

        lax.fori_loop(0, nblk, block, 0)

    run_half(cidxa_v, cdfa_v, cnt0)
    run_half(cidxb_v, cdfb_v, cnt1)

    plsc.subcore_barrier()
    pltpu.sync_copy(zsp.at[pl.ds(sub * RPT, RPT)],
                    zout_hbm.at[core, pl.ds(sub * RPT, RPT)])


_seg_kernel = pl.kernel(
    _seg_body,
    out_type=jax.ShapeDtypeStruct((NC, NPAD, HALF), jnp.float32),
    mesh=plsc.VectorSubcoreMesh(core_axis_name="c", subcore_axis_name="s"),
    scratch_types=[
        pltpu.VMEM((CH,), jnp.int32),
        pltpu.VMEM((CH,), jnp.int32),
        pltpu.VMEM((16,), jnp.int32),
        pltpu.VMEM((16,), jnp.int32),
        pltpu.VMEM((HA + 128,), jnp.int32),
        pltpu.VMEM((HA + 128,), jnp.int32),
        pltpu.VMEM((HB + 128,), jnp.int32),
        pltpu.VMEM((HB + 128,), jnp.int32),
        pltpu.VMEM((2, BLK), jnp.int32),
        pltpu.VMEM((2, BLK, HALF), jnp.float32),
        pltpu.VMEM_SHARED((NPAD, HALF), jnp.float32),
        pltpu.SemaphoreType.DMA,
    ],
    compiler_params=_SC_PARAMS,
)


DEG_NB = CH // BLK + 1


def _deg_body(dst_hbm, zinit1_hbm, degout_hbm, dst_v, idxr_v, ones_v, dsp,
              sem, ssem):
    core = lax.axis_index("c")
    sub = lax.axis_index("s")
    wid = sub * NC + core

    d1 = pltpu.async_copy(dst_hbm.at[pl.ds(wid * CH, CH)], dst_v, sem)
    d2 = pltpu.async_copy(zinit1_hbm, dsp.at[pl.ds(sub * RPT, RPT)], sem)
    for k in range(BLK // 16):
        ones_v[pl.ds(16 * k, 16)] = jnp.ones((16,), jnp.float32)
    d1.wait(); d2.wait()
    plsc.subcore_barrier()

    nfull = CH // BLK

    def wait_scatter(slot):
        pltpu.make_async_copy(ones_v, dsp.at[idxr_v.at[slot]], ssem).wait()

    def block_scatter(j, _):
        slot = lax.rem(j, 4)

        @pl.when(j >= 4)
        def _():
            wait_scatter(slot)

        for k in range(BLK // 16):
            idxr_v[slot, pl.ds(16 * k, 16)] = \
                dst_v[pl.ds(j * BLK + 16 * k, 16)]
        pltpu.async_copy(ones_v, dsp.at[idxr_v.at[slot]], ssem, add=True)
        return 0

    lax.fori_loop(0, nfull, block_scatter, 0)

    slot = lax.rem(jnp.int32(nfull), 4)
    wait_scatter(slot)
    idxr_v[slot, pl.ds(0, 16)] = dst_v[pl.ds(nfull * BLK, 16)]
    for k in range(1, BLK // 16):
        idxr_v[slot, pl.ds(16 * k, 16)] = jnp.full((16,), DUMP, jnp.int32)
    pltpu.async_copy(ones_v, dsp.at[idxr_v.at[slot]], ssem, add=True)

    for k in range(4):
        wait_scatter(jnp.int32(k))

    plsc.subcore_barrier()
    pltpu.sync_copy(dsp.at[pl.ds(sub * RPT, RPT)],
                    degout_hbm.at[core, pl.ds(sub * RPT, RPT)])


_deg_kernel = pl.kernel(
    _deg_body,
    out_type=jax.ShapeDtypeStruct((NC, NPAD), jnp.float32),
    mesh=plsc.VectorSubcoreMesh(core_axis_name="c", subcore_axis_name="s"),
    scratch_types=[
        pltpu.VMEM((CH,), jnp.int32),
        pltpu.VMEM((4, BLK), jnp.int32),
        pltpu.VMEM((BLK,), jnp.float32),
        pltpu.VMEM_SHARED((NPAD,), jnp.float32),
        pltpu.SemaphoreType.DMA,
        pltpu.SemaphoreType.DMA,
    ],
    compiler_params=_SC_PARAMS,
)


PQ = 32


def _po_body(h_hbm, pos_hbm, hg_hbm, idx_v, rows_v, sem):
    core = lax.axis_index("c")
    sub = lax.axis_index("s")
    wid = sub * NC + core
    pltpu.sync_copy(pos_hbm.at[pl.ds(wid * PQ, PQ)], idx_v)
    pltpu.async_copy(h_hbm.at[idx_v], rows_v, sem).wait()
    pltpu.sync_copy(rows_v, hg_hbm.at[pl.ds(wid * PQ, PQ)])


_po_kernel = pl.kernel(
    _po_body,
    out_type=jax.ShapeDtypeStruct((NW * PQ, H), jnp.float32),
    mesh=plsc.VectorSubcoreMesh(core_axis_name="c", subcore_axis_name="s"),
    scratch_types=[
        pltpu.VMEM((PQ,), jnp.int32),
        pltpu.VMEM((PQ, H), jnp.float32),
        pltpu.SemaphoreType.DMA,
    ],
    compiler_params=_SC_PARAMS,
)


BA = 400
NBA = N // BA


def _init_body(feat_r, delay_r, nl_r, ws1_r, bs1_r, ws2_r, bs2_r,
               wp1_r, bp1_r, wp2_r, bp2_r, wn1_r,
               s_o, h_o, g_o):
    x = feat_r[...]
    s1 = _leaky(jnp.dot(x, ws1_r[...], preferred_element_type=jnp.float32)
                + bs1_r[...])
    s_o[...] = jnp.dot(s1, ws2_r[...], preferred_element_type=jnp.float32) \
        + bs2_r[...]
    d = delay_r[...]
    p1 = _leaky(d * wp1_r[...] + bp1_r[...])
    hp = jnp.dot(p1, wp2_r[...], preferred_element_type=jnp.float32) \
        + bp2_r[...]
    m0 = nl_r[...] == 0
    hblk = jnp.where(m0, hp, 0.0)
    h_o[...] = hblk
    g_o[...] = jnp.dot(hblk, wn1_r[...], preferred_element_type=jnp.float32)


def _tc_init(feat, delay, nl2, W_s1, b_s1, W_s2, b_s2,
             W_pi1, b_pi1, W_pi2, b_pi2, W_n1):
    full = lambda i: (0, 0)
    row = lambda i: (i, 0)
    return pl.pallas_call(
        _init_body,
        grid=(NBA,),
        in_specs=[
            pl.BlockSpec((BA, D), row),
            pl.BlockSpec((BA, 1), row),
            pl.BlockSpec((BA, 1), row),
            pl.BlockSpec((D, HALF), full),
            pl.BlockSpec((1, HALF), full),
            pl.BlockSpec((HALF, H), full),
            pl.BlockSpec((1, H), full),
            pl.BlockSpec((1, HALF), full),
            pl.BlockSpec((1, HALF), full),
            pl.BlockSpec((HALF, H), full),
            pl.BlockSpec((1, H), full),
            pl.BlockSpec((H, HALF), full),
        ],
        out_specs=[
            pl.BlockSpec((BA, H), row),
            pl.BlockSpec((BA, H), row),
            pl.BlockSpec((BA, HALF), row),
        ],
        out_shape=[
            jax.ShapeDtypeStruct((N, H), jnp.float32),
            jax.ShapeDtypeStruct((N, H), jnp.float32),
            jax.ShapeDtypeStruct((N, HALF), jnp.float32),
        ],
    )(feat, delay, nl2, W_s1, b_s1, W_s2, b_s2,
      W_pi1, b_pi1, W_pi2, b_pi2, W_n1)


BD = 1000
NBD = N // BD


def _lvl_body(z_r, dg_r, s_r, nl_r, po_r, lvl_r, h_r, g_r,
              wn1_r, bn1_r, wn2_r, bn2_r, h_o, g_o):
    z = z_r[0] + z_r[1]
    deg = jnp.maximum(dg_r[0] + dg_r[1], 1.0)
    zn = z / deg
    a1 = _leaky(zn + bn1_r[...])
    t = jnp.dot(a1, wn2_r[...], preferred_element_type=jnp.float32) \
        + bn2_r[...] + s_r[...]
    t = jnp.where(po_r[...] != 1, jnp.maximum(t, 0.0), t)
    m = nl_r[...] == lvl_r[...]
    h_o[...] = jnp.where(m, t, h_r[...])
    g_o[...] = jnp.where(
        m, jnp.dot(t, wn1_r[...], preferred_element_type=jnp.float32),
        g_r[...])


def _tc_level(zpart, deg3, S, nl2, po2, lvl2, h, g,
              W_n1, b_n1, W_n2, b_n2):
    full = lambda i: (0, 0)
    row = lambda i: (i, 0)
    return pl.pallas_call(
        _lvl_body,
        grid=(NBD,),
        in_specs=[
            pl.BlockSpec((NC, BD, HALF), lambda i: (0, i, 0)),
            pl.BlockSpec((NC, BD, 1), lambda i: (0, i, 0)),
            pl.BlockSpec((BD, H), row),
            pl.BlockSpec((BD, 1), row),
            pl.BlockSpec((BD, 1), row),
            pl.BlockSpec((1, 1), full),
            pl.BlockSpec((BD, H), row),
            pl.BlockSpec((BD, HALF), row),
            pl.BlockSpec((H, HALF), full),
            pl.BlockSpec((1, HALF), full),
            pl.BlockSpec((HALF, H), full),
            pl.BlockSpec((1, H), full),
        ],
        out_specs=[
            pl.BlockSpec((BD, H), row),
            pl.BlockSpec((BD, HALF), row),
        ],
        out_shape=[
            jax.ShapeDtypeStruct((N, H), jnp.float32),
            jax.ShapeDtypeStruct((N, HALF), jnp.float32),
        ],
    )(zpart, deg3, S, nl2, po2, lvl2, h, g, W_n1, b_n1, W_n2, b_n2)


def _fin_body(hg_r, pf_r, wg1_r, bg1_r, wg2_r, bg2_r,
              wo1_r, bo1_r, wo2_r, bo2_r, out_o):
    q1 = _leaky(pf_r[...] * wg1_r[...] + bg1_r[...])
    hglob = jnp.dot(q1, wg2_r[...], preferred_element_type=jnp.float32) \
        + bg2_r[...]
    u = jnp.dot(hg_r[...], wo1_r[0:H, :],
                preferred_element_type=jnp.float32) \
        + jnp.dot(hglob, wo1_r[H:2 * H, :],
                  preferred_element_type=jnp.float32) + bo1_r[...]
    u = _leaky(u)
    out_o[...] = jnp.dot(u, wo2_r[...], preferred_element_type=jnp.float32) \
        + bo2_r[...]


def _tc_final(hg, PO_feat, W_g1, b_g1, W_g2, b_g2, W_o1, b_o1, W_o2, b_o2):
    return pl.pallas_call(
        _fin_body,
        out_shape=jax.ShapeDtypeStruct((P, 1), jnp.float32),
    )(hg, PO_feat, W_g1, b_g1, W_g2, b_g2, W_o1, b_o1, W_o2, b_o2)


def kernel(feat, delay, edge_index, node_level, is_po, POs, PO_feat,
           W_pi1, b_pi1, W_pi2, b_pi2,
           W_n1, b_n1, W_n2, b_n2,
           W_s1, b_s1, W_s2, b_s2,
           W_g1, b_g1, W_g2, b_g2,
           W_o1, b_o1, W_o2, b_o2):
    f32 = jnp.float32
    nl2 = node_level.reshape(N, 1)
    po2 = is_po.reshape(N, 1)
    b_pi1r = b_pi1.reshape(1, HALF)
    b_pi2r = b_pi2.reshape(1, H)
    b_n1r = b_n1.reshape(1, HALF)
    b_n2r = b_n2.reshape(1, H)
    b_s1r = b_s1.reshape(1, HALF)
    b_s2r = b_s2.reshape(1, H)
    b_g1r = b_g1.reshape(1, HALF)
    b_g2r = b_g2.reshape(1, H)
    b_o1r = b_o1.reshape(1, H)
    b_o2r = b_o2.reshape(1, 1)
    W_g1r = W_g1.reshape(1, HALF)
    W_pi1r = W_pi1.reshape(1, HALF)

    zinit = jnp.zeros((RPT, HALF), f32)
    zinit1 = jnp.zeros((RPT,), f32)

    S, h, g = _tc_init(feat, delay, nl2, W_s1, b_s1r, W_s2, b_s2r,
                       W_pi1r, b_pi1r, W_pi2, b_pi2r, W_n1)

    src = edge_index[0]
    dst = edge_index[1]
    degout = _deg_kernel(dst, zinit1)
    deg3 = degout.reshape(NC, NPAD, 1)

    bounds = jnp.searchsorted(node_level, jnp.arange(L + 1, dtype=jnp.int32))
    bounds = bounds.astype(jnp.int32)

    for lvl in range(1, L):
        lvl2 = jnp.full((1, 1), lvl, jnp.int32)
        lo_arr = jnp.broadcast_to(bounds[lvl], (16,))
        hi_arr = jnp.broadcast_to(bounds[lvl + 1], (16,))
        zpart = _seg_kernel(src, dst, g, lo_arr, hi_arr, zinit)
        h, g = _tc_level(zpart, deg3, S, nl2, po2, lvl2, h, g,
                         W_n1, b_n1r, W_n2, b_n2r)

    pos_pad = jnp.concatenate(
        [POs, jnp.zeros((NW * PQ - P,), jnp.int32)])
    hg_pad = _po_kernel(h, pos_pad)
    hg = hg_pad[:P]

    return _tc_final(hg, PO_feat, W_g1r, b_g1r, W_g2, b_g2r,
                     W_o1, b_o1r, W_o2, b_o2r)


# speedup vs baseline: 1.1864x; 1.1864x over previous
import functools

import jax
import jax.numpy as jnp
from jax import lax
from jax.experimental import pallas as pl
from jax.experimental.pallas import tpu as pltpu
from jax.experimental.pallas import tpu_sc as plsc

_SC_PARAMS = pltpu.CompilerParams(needs_layout_passes=False,
                                  use_tc_tiling_on_sc=False)

N = 10000
E = 320000
D = 128
H = 128
HALF = 64
L = 8
P = 1000

NC = 2
NS = 16
NW = NC * NS
CH = E // NW
NSTEP = CH // 16
BLK = 128
NPAD = 10240
RPT = NPAD // NS
DUMP = N


def _leaky(x):
    return jnp.where(x > 0, x, 0.1 * x)


HA = 5008
HB = CH - HA


def _seg_body(src_hbm, dst_hbm, g_hbm, lo_hbm, hi_hbm, zinit_hbm, zout_hbm,
              src_v, dst_v, lo_v, hi_v, cidxa_v, cdfa_v,
              cd2_v, rows_v, zsp, gsem):
    core = lax.axis_index("c")
    sub = lax.axis_index("s")
    wid = sub * NC + core

    d2 = pltpu.async_copy(src_hbm.at[pl.ds(wid * CH, CH)], src_v, gsem)
    d3 = pltpu.async_copy(dst_hbm.at[pl.ds(wid * CH, CH)], dst_v, gsem)
    d4 = pltpu.async_copy(lo_hbm, lo_v, gsem)
    d5 = pltpu.async_copy(hi_hbm, hi_v, gsem)
    d6 = pltpu.async_copy(zinit_hbm, zsp.at[pl.ds(sub * RPT, RPT)], gsem)
    d2.wait(); d3.wait(); d4.wait(); d5.wait(); d6.wait()
    plsc.subcore_barrier()

    lo = lo_v[...]
    hi = hi_v[...]

    def gate(i, cnt):
        da = dst_v[pl.ds(i * 16, 16)]
        sa = src_v[pl.ds(i * 16, 16)]
        ma = (da >= lo) & (da < hi)
        plsc.store_compressed(cidxa_v.at[pl.ds(cnt, 16)], sa, mask=ma)
        plsc.store_compressed(cdfa_v.at[pl.ds(cnt, 16)], da, mask=ma)
        return cnt + jnp.sum(ma.astype(jnp.int32))

    cnt0 = lax.fori_loop(0, NSTEP, gate, jnp.int32(0))

    for k in range(8):
        cidxa_v[pl.ds(cnt0 + 16 * k, 16)] = jnp.zeros((16,), jnp.int32)
        cdfa_v[pl.ds(cnt0 + 16 * k, 16)] = jnp.full((16,), DUMP, jnp.int32)

    def run_half(cidx_v, cdf_v, cnt):
        nblk = (cnt + (BLK - 1)) // BLK

        def start_gather(j, p):
            pltpu.async_copy(g_hbm.at[cidx_v.at[pl.ds(j * BLK, BLK)]],
                             rows_v.at[p], gsem)

        @pl.when(nblk > 0)
        def _():
            start_gather(0, 0)

        def block(j, _):
            p = lax.rem(j, 2)
            pltpu.make_async_copy(
                g_hbm.at[cidx_v.at[pl.ds(j * BLK, BLK)]],
                rows_v.at[p], gsem).wait()

            @pl.when(j + 1 < nblk)
            def _():
                start_gather(j + 1, 1 - p)

            for k in range(BLK // 16):
                cd2_v[p, pl.ds(16 * k, 16)] = \
                    cdf_v[pl.ds(j * BLK + 16 * k, 16)]
            pltpu.sync_copy(rows_v.at[p], zsp.at[cd2_v.at[p]], add=True)
            return 0

        lax.fori_loop(0, nblk, block, 0)

    run_half(cidxa_v, cdfa_v, cnt0)

    plsc.subcore_barrier()
    pltpu.sync_copy(zsp.at[pl.ds(sub * RPT, RPT)],
                    zout_hbm.at[core, pl.ds(sub * RPT, RPT)])


_seg_kernel = pl.kernel(
    _seg_body,
    out_type=jax.ShapeDtypeStruct((NC, NPAD, HALF), jnp.float32),
    mesh=plsc.VectorSubcoreMesh(core_axis_name="c", subcore_axis_name="s"),
    scratch_types=[
        pltpu.VMEM((CH,), jnp.int32),
        pltpu.VMEM((CH,), jnp.int32),
        pltpu.VMEM((16,), jnp.int32),
        pltpu.VMEM((16,), jnp.int32),
        pltpu.VMEM((CH + 128,), jnp.int32),
        pltpu.VMEM((CH + 128,), jnp.int32),
        pltpu.VMEM((2, BLK), jnp.int32),
        pltpu.VMEM((2, BLK, HALF), jnp.float32),
        pltpu.VMEM_SHARED((NPAD, HALF), jnp.float32),
        pltpu.SemaphoreType.DMA,
    ],
    compiler_params=_SC_PARAMS,
)


DEG_NB = CH // BLK + 1


def _deg_body(dst_hbm, zinit1_hbm, degout_hbm, dst_v, idxr_v, ones_v, dsp,
              sem, ssem):
    core = lax.axis_index("c")
    sub = lax.axis_index("s")
    wid = sub * NC + core

    d1 = pltpu.async_copy(dst_hbm.at[pl.ds(wid * CH, CH)], dst_v, sem)
    d2 = pltpu.async_copy(zinit1_hbm, dsp.at[pl.ds(sub * RPT, RPT)], sem)
    for k in range(BLK // 16):
        ones_v[pl.ds(16 * k, 16)] = jnp.ones((16,), jnp.float32)
    d1.wait(); d2.wait()
    plsc.subcore_barrier()

    nfull = CH // BLK

    def wait_scatter(slot):
        pltpu.make_async_copy(ones_v, dsp.at[idxr_v.at[slot]], ssem).wait()

    def block_scatter(j, _):
        slot = lax.rem(j, 4)

        @pl.when(j >= 4)
        def _():
            wait_scatter(slot)

        for k in range(BLK // 16):
            idxr_v[slot, pl.ds(16 * k, 16)] = \
                dst_v[pl.ds(j * BLK + 16 * k, 16)]
        pltpu.async_copy(ones_v, dsp.at[idxr_v.at[slot]], ssem, add=True)
        return 0

    lax.fori_loop(0, nfull, block_scatter, 0)

    slot = lax.rem(jnp.int32(nfull), 4)
    wait_scatter(slot)
    idxr_v[slot, pl.ds(0, 16)] = dst_v[pl.ds(nfull * BLK, 16)]
    for k in range(1, BLK // 16):
        idxr_v[slot, pl.ds(16 * k, 16)] = jnp.full((16,), DUMP, jnp.int32)
    pltpu.async_copy(ones_v, dsp.at[idxr_v.at[slot]], ssem, add=True)

    for k in range(4):
        wait_scatter(jnp.int32(k))

    plsc.subcore_barrier()
    pltpu.sync_copy(dsp.at[pl.ds(sub * RPT, RPT)],
                    degout_hbm.at[core, pl.ds(sub * RPT, RPT)])


_deg_kernel = pl.kernel(
    _deg_body,
    out_type=jax.ShapeDtypeStruct((NC, NPAD), jnp.float32),
    mesh=plsc.VectorSubcoreMesh(core_axis_name="c", subcore_axis_name="s"),
    scratch_types=[
        pltpu.VMEM((CH,), jnp.int32),
        pltpu.VMEM((4, BLK), jnp.int32),
        pltpu.VMEM((BLK,), jnp.float32),
        pltpu.VMEM_SHARED((NPAD,), jnp.float32),
        pltpu.SemaphoreType.DMA,
        pltpu.SemaphoreType.DMA,
    ],
    compiler_params=_SC_PARAMS,
)


PQ = 32


def _po_body(h_hbm, pos_hbm, hg_hbm, idx_v, rows_v, sem):
    core = lax.axis_index("c")
    sub = lax.axis_index("s")
    wid = sub * NC + core
    pltpu.sync_copy(pos_hbm.at[pl.ds(wid * PQ, PQ)], idx_v)
    pltpu.async_copy(h_hbm.at[idx_v], rows_v, sem).wait()
    pltpu.sync_copy(rows_v, hg_hbm.at[pl.ds(wid * PQ, PQ)])


_po_kernel = pl.kernel(
    _po_body,
    out_type=jax.ShapeDtypeStruct((NW * PQ, H), jnp.float32),
    mesh=plsc.VectorSubcoreMesh(core_axis_name="c", subcore_axis_name="s"),
    scratch_types=[
        pltpu.VMEM((PQ,), jnp.int32),
        pltpu.VMEM((PQ, H), jnp.float32),
        pltpu.SemaphoreType.DMA,
    ],
    compiler_params=_SC_PARAMS,
)


BA = 400
NBA = N // BA


def _init_body(feat_r, delay_r, nl_r, ws1_r, bs1_r, ws2_r, bs2_r,
               wp1_r, bp1_r, wp2_r, bp2_r, wn1_r,
               s_o, h_o, g_o):
    x = feat_r[...]
    s1 = _leaky(jnp.dot(x, ws1_r[...], preferred_element_type=jnp.float32)
                + bs1_r[...])
    s_o[...] = jnp.dot(s1, ws2_r[...], preferred_element_type=jnp.float32) \
        + bs2_r[...]
    d = delay_r[...]
    p1 = _leaky(d * wp1_r[...] + bp1_r[...])
    hp = jnp.dot(p1, wp2_r[...], preferred_element_type=jnp.float32) \
        + bp2_r[...]
    m0 = nl_r[...] == 0
    hblk = jnp.where(m0, hp, 0.0)
    h_o[...] = hblk
    g_o[...] = jnp.dot(hblk, wn1_r[...], preferred_element_type=jnp.float32)


def _tc_init(feat, delay, nl2, W_s1, b_s1, W_s2, b_s2,
             W_pi1, b_pi1, W_pi2, b_pi2, W_n1):
    full = lambda i: (0, 0)
    row = lambda i: (i, 0)
    return pl.pallas_call(
        _init_body,
        grid=(NBA,),
        in_specs=[
            pl.BlockSpec((BA, D), row),
            pl.BlockSpec((BA, 1), row),
            pl.BlockSpec((BA, 1), row),
            pl.BlockSpec((D, HALF), full),
            pl.BlockSpec((1, HALF), full),
            pl.BlockSpec((HALF, H), full),
            pl.BlockSpec((1, H), full),
            pl.BlockSpec((1, HALF), full),
            pl.BlockSpec((1, HALF), full),
            pl.BlockSpec((HALF, H), full),
            pl.BlockSpec((1, H), full),
            pl.BlockSpec((H, HALF), full),
        ],
        out_specs=[
            pl.BlockSpec((BA, H), row),
            pl.BlockSpec((BA, H), row),
            pl.BlockSpec((BA, HALF), row),
        ],
        out_shape=[
            jax.ShapeDtypeStruct((N, H), jnp.float32),
            jax.ShapeDtypeStruct((N, H), jnp.float32),
            jax.ShapeDtypeStruct((N, HALF), jnp.float32),
        ],
    )(feat, delay, nl2, W_s1, b_s1, W_s2, b_s2,
      W_pi1, b_pi1, W_pi2, b_pi2, W_n1)


BD = 1000
NBD = N // BD


def _lvl_body(z_r, dg_r, s_r, nl_r, po_r, lvl_r, h_r, g_r,
              wn1_r, bn1_r, wn2_r, bn2_r, h_o, g_o):
    z = z_r[0] + z_r[1]
    deg = jnp.maximum(dg_r[0] + dg_r[1], 1.0)
    zn = z / deg
    a1 = _leaky(zn + bn1_r[...])
    t = jnp.dot(a1, wn2_r[...], preferred_element_type=jnp.float32) \
        + bn2_r[...] + s_r[...]
    t = jnp.where(po_r[...] != 1, jnp.maximum(t, 0.0), t)
    m = nl_r[...] == lvl_r[...]
    h_o[...] = jnp.where(m, t, h_r[...])
    g_o[...] = jnp.where(
        m, jnp.dot(t, wn1_r[...], preferred_element_type=jnp.float32),
        g_r[...])


def _tc_level(zpart, deg3, S, nl2, po2, lvl2, h, g,
              W_n1, b_n1, W_n2, b_n2):
    full = lambda i: (0, 0)
    row = lambda i: (i, 0)
    return pl.pallas_call(
        _lvl_body,
        grid=(NBD,),
        in_specs=[
            pl.BlockSpec((NC, BD, HALF), lambda i: (0, i, 0)),
            pl.BlockSpec((NC, BD, 1), lambda i: (0, i, 0)),
            pl.BlockSpec((BD, H), row),
            pl.BlockSpec((BD, 1), row),
            pl.BlockSpec((BD, 1), row),
            pl.BlockSpec((1, 1), full),
            pl.BlockSpec((BD, H), row),
            pl.BlockSpec((BD, HALF), row),
            pl.BlockSpec((H, HALF), full),
            pl.BlockSpec((1, HALF), full),
            pl.BlockSpec((HALF, H), full),
            pl.BlockSpec((1, H), full),
        ],
        out_specs=[
            pl.BlockSpec((BD, H), row),
            pl.BlockSpec((BD, HALF), row),
        ],
        out_shape=[
            jax.ShapeDtypeStruct((N, H), jnp.float32),
            jax.ShapeDtypeStruct((N, HALF), jnp.float32),
        ],
    )(zpart, deg3, S, nl2, po2, lvl2, h, g, W_n1, b_n1, W_n2, b_n2)


def _fin_body(hg_r, pf_r, wg1_r, bg1_r, wg2_r, bg2_r,
              wo1_r, bo1_r, wo2_r, bo2_r, out_o):
    q1 = _leaky(pf_r[...] * wg1_r[...] + bg1_r[...])
    hglob = jnp.dot(q1, wg2_r[...], preferred_element_type=jnp.float32) \
        + bg2_r[...]
    u = jnp.dot(hg_r[...], wo1_r[0:H, :],
                preferred_element_type=jnp.float32) \
        + jnp.dot(hglob, wo1_r[H:2 * H, :],
                  preferred_element_type=jnp.float32) + bo1_r[...]
    u = _leaky(u)
    out_o[...] = jnp.dot(u, wo2_r[...], preferred_element_type=jnp.float32) \
        + bo2_r[...]


def _tc_final(hg, PO_feat, W_g1, b_g1, W_g2, b_g2, W_o1, b_o1, W_o2, b_o2):
    return pl.pallas_call(
        _fin_body,
        out_shape=jax.ShapeDtypeStruct((P, 1), jnp.float32),
    )(hg, PO_feat, W_g1, b_g1, W_g2, b_g2, W_o1, b_o1, W_o2, b_o2)


def kernel(feat, delay, edge_index, node_level, is_po, POs, PO_feat,
           W_pi1, b_pi1, W_pi2, b_pi2,
           W_n1, b_n1, W_n2, b_n2,
           W_s1, b_s1, W_s2, b_s2,
           W_g1, b_g1, W_g2, b_g2,
           W_o1, b_o1, W_o2, b_o2):
    f32 = jnp.float32
    nl2 = node_level.reshape(N, 1)
    po2 = is_po.reshape(N, 1)
    b_pi1r = b_pi1.reshape(1, HALF)
    b_pi2r = b_pi2.reshape(1, H)
    b_n1r = b_n1.reshape(1, HALF)
    b_n2r = b_n2.reshape(1, H)
    b_s1r = b_s1.reshape(1, HALF)
    b_s2r = b_s2.reshape(1, H)
    b_g1r = b_g1.reshape(1, HALF)
    b_g2r = b_g2.reshape(1, H)
    b_o1r = b_o1.reshape(1, H)
    b_o2r = b_o2.reshape(1, 1)
    W_g1r = W_g1.reshape(1, HALF)
    W_pi1r = W_pi1.reshape(1, HALF)

    zinit = jnp.zeros((RPT, HALF), f32)
    zinit1 = jnp.zeros((RPT,), f32)

    S, h, g = _tc_init(feat, delay, nl2, W_s1, b_s1r, W_s2, b_s2r,
                       W_pi1r, b_pi1r, W_pi2, b_pi2r, W_n1)

    src = edge_index[0]
    dst = edge_index[1]
    degout = _deg_kernel(dst, zinit1)
    deg3 = degout.reshape(NC, NPAD, 1)

    bounds = jnp.searchsorted(node_level, jnp.arange(L + 1, dtype=jnp.int32))
    bounds = bounds.astype(jnp.int32)

    for lvl in range(1, L):
        lvl2 = jnp.full((1, 1), lvl, jnp.int32)
        lo_arr = jnp.broadcast_to(bounds[lvl], (16,))
        hi_arr = jnp.broadcast_to(bounds[lvl + 1], (16,))
        zpart = _seg_kernel(src, dst, g, lo_arr, hi_arr, zinit)
        h, g = _tc_level(zpart, deg3, S, nl2, po2, lvl2, h, g,
                         W_n1, b_n1r, W_n2, b_n2r)

    pos_pad = jnp.concatenate(
        [POs, jnp.zeros((NW * PQ - P,), jnp.int32)])
    hg_pad = _po_kernel(h, pos_pad)
    hg = hg_pad[:P]

    return _tc_final(hg, PO_feat, W_g1r, b_g1r, W_g2, b_g2r,
                     W_o1, b_o1r, W_o2, b_o2r)


# speedup vs baseline: 1.2759x; 1.0755x over previous
import functools

import jax
import jax.numpy as jnp
from jax import lax
from jax.experimental import pallas as pl
from jax.experimental.pallas import tpu as pltpu
from jax.experimental.pallas import tpu_sc as plsc

_SC_PARAMS = pltpu.CompilerParams(needs_layout_passes=False,
                                  use_tc_tiling_on_sc=False)

N = 10000
E = 320000
D = 128
H = 128
HALF = 64
L = 8
P = 1000

NC = 2
NS = 16
NW = NC * NS
CH = E // NW
NSTEP = CH // 16
BLK = 128
NPAD = 10240
RPT = NPAD // NS
DUMP = N


def _leaky(x):
    return jnp.where(x > 0, x, 0.1 * x)


ASZ = CH + 1024


def _prep_body(src_hbm, dst_hbm, bnd_hbm, asrc_hbm, adst_hbm, meta_hbm,
               src_v, dst_v, bnd_v, asrc_v, adst_v, meta_v, sem):
    core = lax.axis_index("c")
    sub = lax.axis_index("s")
    wid = sub * NC + core

    d1 = pltpu.async_copy(src_hbm.at[pl.ds(wid * CH, CH)], src_v, sem)
    d2 = pltpu.async_copy(dst_hbm.at[pl.ds(wid * CH, CH)], dst_v, sem)
    d3 = pltpu.async_copy(bnd_hbm, bnd_v, sem)
    d1.wait(); d2.wait(); d3.wait()

    bnds = [bnd_v[l] for l in range(8)]

    def _lev(d):
        lev = (d >= bnds[0]).astype(jnp.int32)
        for l in range(1, 8):
            lev = lev + (d >= bnds[l]).astype(jnp.int32)
        return lev

    def count_step(i, accs):
        d = dst_v[pl.ds(i * 16, 16)]
        lev = _lev(d)
        return tuple(accs[l - 1] + (lev == l).astype(jnp.int32)
                     for l in range(1, 8))

    accs = lax.fori_loop(0, NSTEP, count_step,
                         tuple(jnp.zeros((16,), jnp.int32)
                               for _ in range(7)))
    cnts = [jnp.sum(a) for a in accs]

    bases = []
    b = jnp.int32(0)
    for l in range(7):
        bases.append(b)
        b = b + ((cnts[l] + (BLK - 1)) // BLK) * BLK

    for l in range(7):
        for k in range(8):
            asrc_v[pl.ds(bases[l] + cnts[l] + 16 * k, 16)] = \
                jnp.zeros((16,), jnp.int32)
            adst_v[pl.ds(bases[l] + cnts[l] + 16 * k, 16)] = \
                jnp.full((16,), DUMP, jnp.int32)

    def compact_step(i, cs):
        d = dst_v[pl.ds(i * 16, 16)]
        s = src_v[pl.ds(i * 16, 16)]
        lev = _lev(d)
        out = []
        for l in range(1, 8):
            m = lev == l
            c = cs[l - 1]
            plsc.store_compressed(asrc_v.at[pl.ds(c, 16)], s, mask=m)
            plsc.store_compressed(adst_v.at[pl.ds(c, 16)], d, mask=m)
            out.append(c + jnp.sum(m.astype(jnp.int32)))
        return tuple(out)

    lax.fori_loop(0, NSTEP, compact_step, tuple(bases))

    for l in range(7):
        meta_v[l] = jnp.broadcast_to(bases[l], (16,))
        meta_v[l + 7] = jnp.broadcast_to(cnts[l], (16,))

    o1 = pltpu.async_copy(asrc_v, asrc_hbm.at[wid], sem)
    o2 = pltpu.async_copy(adst_v, adst_hbm.at[wid], sem)
    o3 = pltpu.async_copy(meta_v, meta_hbm.at[wid], sem)
    o1.wait(); o2.wait(); o3.wait()


_prep_kernel = pl.kernel(
    _prep_body,
    out_type=(jax.ShapeDtypeStruct((NW, ASZ), jnp.int32),
              jax.ShapeDtypeStruct((NW, ASZ), jnp.int32),
              jax.ShapeDtypeStruct((NW, 16, 16), jnp.int32)),
    mesh=plsc.VectorSubcoreMesh(core_axis_name="c", subcore_axis_name="s"),
    scratch_types=[
        pltpu.VMEM((CH,), jnp.int32),
        pltpu.VMEM((CH,), jnp.int32),
        pltpu.VMEM((8, 16), jnp.int32),
        pltpu.VMEM((ASZ,), jnp.int32),
        pltpu.VMEM((ASZ,), jnp.int32),
        pltpu.VMEM((16, 16), jnp.int32),
        pltpu.SemaphoreType.DMA,
    ],
    compiler_params=_SC_PARAMS,
)


def _seg_body(asrc_hbm, adst_hbm, meta_hbm, g_hbm, lvl_hbm, zinit_hbm,
              zout_hbm, meta_v, lvl_v, bidx_v, cd2_v, rows_v, zsp,
              isem, gsem):
    core = lax.axis_index("c")
    sub = lax.axis_index("s")
    wid = sub * NC + core

    d1 = pltpu.async_copy(meta_hbm.at[wid], meta_v, isem)
    d2 = pltpu.async_copy(lvl_hbm, lvl_v, isem)
    d3 = pltpu.async_copy(zinit_hbm, zsp.at[pl.ds(sub * RPT, RPT)], isem)
    d1.wait(); d2.wait(); d3.wait()
    plsc.subcore_barrier()

    l = lvl_v[...][0]
    base = pl.multiple_of(meta_v[l - 1][0], BLK)
    cnt = meta_v[l + 6][0]
    nblk = (cnt + (BLK - 1)) // BLK

    def stage(j, p):
        pltpu.async_copy(asrc_hbm.at[wid, pl.ds(base + j * BLK, BLK)],
                         bidx_v.at[p], isem)
        pltpu.async_copy(adst_hbm.at[wid, pl.ds(base + j * BLK, BLK)],
                         cd2_v.at[p], isem)

    def wait_stage(j, p):
        pltpu.make_async_copy(asrc_hbm.at[wid, pl.ds(base + j * BLK, BLK)],
                              bidx_v.at[p], isem).wait()
        pltpu.make_async_copy(adst_hbm.at[wid, pl.ds(base + j * BLK, BLK)],
                              cd2_v.at[p], isem).wait()

    def gather(j, p):
        pltpu.async_copy(g_hbm.at[bidx_v.at[p]], rows_v.at[p], gsem)

    def wait_gather(j, p):
        pltpu.make_async_copy(g_hbm.at[bidx_v.at[p]], rows_v.at[p],
                              gsem).wait()

    @pl.when(nblk > 0)
    def _():
        stage(0, 0)
        wait_stage(0, 0)
        gather(0, 0)

    def block(j, _):
        p = lax.rem(j, 2)

        @pl.when(j + 1 < nblk)
        def _():
            stage(j + 1, 1 - p)

        wait_gather(j, p)

        @pl.when(j + 1 < nblk)
        def _():
            wait_stage(j + 1, 1 - p)
            gather(j + 1, 1 - p)

        pltpu.sync_copy(rows_v.at[p], zsp.at[cd2_v.at[p]], add=True)
        return 0

    lax.fori_loop(0, nblk, block, 0)

    plsc.subcore_barrier()
    pltpu.sync_copy(zsp.at[pl.ds(sub * RPT, RPT)],
                    zout_hbm.at[core, pl.ds(sub * RPT, RPT)])


_seg_kernel = pl.kernel(
    _seg_body,
    out_type=jax.ShapeDtypeStruct((NC, NPAD, HALF), jnp.float32),
    mesh=plsc.VectorSubcoreMesh(core_axis_name="c", subcore_axis_name="s"),
    scratch_types=[
        pltpu.VMEM((16, 16), jnp.int32),
        pltpu.VMEM((16,), jnp.int32),
        pltpu.VMEM((2, BLK), jnp.int32),
        pltpu.VMEM((2, BLK), jnp.int32),
        pltpu.VMEM((2, BLK, HALF), jnp.float32),
        pltpu.VMEM_SHARED((NPAD, HALF), jnp.float32),
        pltpu.SemaphoreType.DMA,
        pltpu.SemaphoreType.DMA,
    ],
    compiler_params=_SC_PARAMS,
)


DEG_NB = CH // BLK + 1


def _deg_body(dst_hbm, zinit1_hbm, degout_hbm, dst_v, idxr_v, ones_v, dsp,
              sem, ssem):
    core = lax.axis_index("c")
    sub = lax.axis_index("s")
    wid = sub * NC + core

    d1 = pltpu.async_copy(dst_hbm.at[pl.ds(wid * CH, CH)], dst_v, sem)
    d2 = pltpu.async_copy(zinit1_hbm, dsp.at[pl.ds(sub * RPT, RPT)], sem)
    for k in range(BLK // 16):
        ones_v[pl.ds(16 * k, 16)] = jnp.ones((16,), jnp.float32)
    d1.wait(); d2.wait()
    plsc.subcore_barrier()

    nfull = CH // BLK

    def wait_scatter(slot):
        pltpu.make_async_copy(ones_v, dsp.at[idxr_v.at[slot]], ssem).wait()

    def block_scatter(j, _):
        slot = lax.rem(j, 4)

        @pl.when(j >= 4)
        def _():
            wait_scatter(slot)

        for k in range(BLK // 16):
            idxr_v[slot, pl.ds(16 * k, 16)] = \
                dst_v[pl.ds(j * BLK + 16 * k, 16)]
        pltpu.async_copy(ones_v, dsp.at[idxr_v.at[slot]], ssem, add=True)
        return 0

    lax.fori_loop(0, nfull, block_scatter, 0)

    slot = lax.rem(jnp.int32(nfull), 4)
    wait_scatter(slot)
    idxr_v[slot, pl.ds(0, 16)] = dst_v[pl.ds(nfull * BLK, 16)]
    for k in range(1, BLK // 16):
        idxr_v[slot, pl.ds(16 * k, 16)] = jnp.full((16,), DUMP, jnp.int32)
    pltpu.async_copy(ones_v, dsp.at[idxr_v.at[slot]], ssem, add=True)

    for k in range(4):
        wait_scatter(jnp.int32(k))

    plsc.subcore_barrier()
    pltpu.sync_copy(dsp.at[pl.ds(sub * RPT, RPT)],
                    degout_hbm.at[core, pl.ds(sub * RPT, RPT)])


_deg_kernel = pl.kernel(
    _deg_body,
    out_type=jax.ShapeDtypeStruct((NC, NPAD), jnp.float32),
    mesh=plsc.VectorSubcoreMesh(core_axis_name="c", subcore_axis_name="s"),
    scratch_types=[
        pltpu.VMEM((CH,), jnp.int32),
        pltpu.VMEM((4, BLK), jnp.int32),
        pltpu.VMEM((BLK,), jnp.float32),
        pltpu.VMEM_SHARED((NPAD,), jnp.float32),
        pltpu.SemaphoreType.DMA,
        pltpu.SemaphoreType.DMA,
    ],
    compiler_params=_SC_PARAMS,
)


PQ = 32


def _po_body(h_hbm, pos_hbm, hg_hbm, idx_v, rows_v, sem):
    core = lax.axis_index("c")
    sub = lax.axis_index("s")
    wid = sub * NC + core
    pltpu.sync_copy(pos_hbm.at[pl.ds(wid * PQ, PQ)], idx_v)
    pltpu.async_copy(h_hbm.at[idx_v], rows_v, sem).wait()
    pltpu.sync_copy(rows_v, hg_hbm.at[pl.ds(wid * PQ, PQ)])


_po_kernel = pl.kernel(
    _po_body,
    out_type=jax.ShapeDtypeStruct((NW * PQ, H), jnp.float32),
    mesh=plsc.VectorSubcoreMesh(core_axis_name="c", subcore_axis_name="s"),
    scratch_types=[
        pltpu.VMEM((PQ,), jnp.int32),
        pltpu.VMEM((PQ, H), jnp.float32),
        pltpu.SemaphoreType.DMA,
    ],
    compiler_params=_SC_PARAMS,
)


BA = 400
NBA = N // BA


def _init_body(feat_r, delay_r, nl_r, ws1_r, bs1_r, ws2_r, bs2_r,
               wp1_r, bp1_r, wp2_r, bp2_r, wn1_r,
               s_o, h_o, g_o):
    x = feat_r[...]
    s1 = _leaky(jnp.dot(x, ws1_r[...], preferred_element_type=jnp.float32)
                + bs1_r[...])
    s_o[...] = jnp.dot(s1, ws2_r[...], preferred_element_type=jnp.float32) \
        + bs2_r[...]
    d = delay_r[...]
    p1 = _leaky(d * wp1_r[...] + bp1_r[...])
    hp = jnp.dot(p1, wp2_r[...], preferred_element_type=jnp.float32) \
        + bp2_r[...]
    m0 = nl_r[...] == 0
    hblk = jnp.where(m0, hp, 0.0)
    h_o[...] = hblk
    g_o[...] = jnp.dot(hblk, wn1_r[...], preferred_element_type=jnp.float32)


def _tc_init(feat, delay, nl2, W_s1, b_s1, W_s2, b_s2,
             W_pi1, b_pi1, W_pi2, b_pi2, W_n1):
    full = lambda i: (0, 0)
    row = lambda i: (i, 0)
    return pl.pallas_call(
        _init_body,
        grid=(NBA,),
        in_specs=[
            pl.BlockSpec((BA, D), row),
            pl.BlockSpec((BA, 1), row),
            pl.BlockSpec((BA, 1), row),
            pl.BlockSpec((D, HALF), full),
            pl.BlockSpec((1, HALF), full),
            pl.BlockSpec((HALF, H), full),
            pl.BlockSpec((1, H), full),
            pl.BlockSpec((1, HALF), full),
            pl.BlockSpec((1, HALF), full),
            pl.BlockSpec((HALF, H), full),
            pl.BlockSpec((1, H), full),
            pl.BlockSpec((H, HALF), full),
        ],
        out_specs=[
            pl.BlockSpec((BA, H), row),
            pl.BlockSpec((BA, H), row),
            pl.BlockSpec((BA, HALF), row),
        ],
        out_shape=[
            jax.ShapeDtypeStruct((N, H), jnp.float32),
            jax.ShapeDtypeStruct((N, H), jnp.float32),
            jax.ShapeDtypeStruct((N, HALF), jnp.float32),
        ],
    )(feat, delay, nl2, W_s1, b_s1, W_s2, b_s2,
      W_pi1, b_pi1, W_pi2, b_pi2, W_n1)


BD = 1000
NBD = N // BD


def _lvl_body(z_r, dg_r, s_r, nl_r, po_r, lvl_r, h_r, g_r,
              wn1_r, bn1_r, wn2_r, bn2_r, h_o, g_o):
    z = z_r[0] + z_r[1]
    deg = jnp.maximum(dg_r[0] + dg_r[1], 1.0)
    zn = z / deg
    a1 = _leaky(zn + bn1_r[...])
    t = jnp.dot(a1, wn2_r[...], preferred_element_type=jnp.float32) \
        + bn2_r[...] + s_r[...]
    t = jnp.where(po_r[...] != 1, jnp.maximum(t, 0.0), t)
    m = nl_r[...] == lvl_r[...]
    h_o[...] = jnp.where(m, t, h_r[...])
    g_o[...] = jnp.where(
        m, jnp.dot(t, wn1_r[...], preferred_element_type=jnp.float32),
        g_r[...])


def _tc_level(zpart, deg3, S, nl2, po2, lvl2, h, g,
              W_n1, b_n1, W_n2, b_n2):
    full = lambda i: (0, 0)
    row = lambda i: (i, 0)
    return pl.pallas_call(
        _lvl_body,
        grid=(NBD,),
        in_specs=[
            pl.BlockSpec((NC, BD, HALF), lambda i: (0, i, 0)),
            pl.BlockSpec((NC, BD, 1), lambda i: (0, i, 0)),
            pl.BlockSpec((BD, H), row),
            pl.BlockSpec((BD, 1), row),
            pl.BlockSpec((BD, 1), row),
            pl.BlockSpec((1, 1), full),
            pl.BlockSpec((BD, H), row),
            pl.BlockSpec((BD, HALF), row),
            pl.BlockSpec((H, HALF), full),
            pl.BlockSpec((1, HALF), full),
            pl.BlockSpec((HALF, H), full),
            pl.BlockSpec((1, H), full),
        ],
        out_specs=[
            pl.BlockSpec((BD, H), row),
            pl.BlockSpec((BD, HALF), row),
        ],
        out_shape=[
            jax.ShapeDtypeStruct((N, H), jnp.float32),
            jax.ShapeDtypeStruct((N, HALF), jnp.float32),
        ],
    )(zpart, deg3, S, nl2, po2, lvl2, h, g, W_n1, b_n1, W_n2, b_n2)


def _fin_body(hg_r, pf_r, wg1_r, bg1_r, wg2_r, bg2_r,
              wo1_r, bo1_r, wo2_r, bo2_r, out_o):
    q1 = _leaky(pf_r[...] * wg1_r[...] + bg1_r[...])
    hglob = jnp.dot(q1, wg2_r[...], preferred_element_type=jnp.float32) \
        + bg2_r[...]
    u = jnp.dot(hg_r[...], wo1_r[0:H, :],
                preferred_element_type=jnp.float32) \
        + jnp.dot(hglob, wo1_r[H:2 * H, :],
                  preferred_element_type=jnp.float32) + bo1_r[...]
    u = _leaky(u)
    out_o[...] = jnp.dot(u, wo2_r[...], preferred_element_type=jnp.float32) \
        + bo2_r[...]


def _tc_final(hg, PO_feat, W_g1, b_g1, W_g2, b_g2, W_o1, b_o1, W_o2, b_o2):
    return pl.pallas_call(
        _fin_body,
        out_shape=jax.ShapeDtypeStruct((P, 1), jnp.float32),
    )(hg, PO_feat, W_g1, b_g1, W_g2, b_g2, W_o1, b_o1, W_o2, b_o2)


def kernel(feat, delay, edge_index, node_level, is_po, POs, PO_feat,
           W_pi1, b_pi1, W_pi2, b_pi2,
           W_n1, b_n1, W_n2, b_n2,
           W_s1, b_s1, W_s2, b_s2,
           W_g1, b_g1, W_g2, b_g2,
           W_o1, b_o1, W_o2, b_o2):
    f32 = jnp.float32
    nl2 = node_level.reshape(N, 1)
    po2 = is_po.reshape(N, 1)
    b_pi1r = b_pi1.reshape(1, HALF)
    b_pi2r = b_pi2.reshape(1, H)
    b_n1r = b_n1.reshape(1, HALF)
    b_n2r = b_n2.reshape(1, H)
    b_s1r = b_s1.reshape(1, HALF)
    b_s2r = b_s2.reshape(1, H)
    b_g1r = b_g1.reshape(1, HALF)
    b_g2r = b_g2.reshape(1, H)
    b_o1r = b_o1.reshape(1, H)
    b_o2r = b_o2.reshape(1, 1)
    W_g1r = W_g1.reshape(1, HALF)
    W_pi1r = W_pi1.reshape(1, HALF)

    zinit = jnp.zeros((RPT, HALF), f32)
    zinit1 = jnp.zeros((RPT,), f32)

    S, h, g = _tc_init(feat, delay, nl2, W_s1, b_s1r, W_s2, b_s2r,
                       W_pi1r, b_pi1r, W_pi2, b_pi2r, W_n1)

    src = edge_index[0]
    dst = edge_index[1]
    degout = _deg_kernel(dst, zinit1)
    deg3 = degout.reshape(NC, NPAD, 1)

    bounds = jnp.searchsorted(node_level, jnp.arange(L + 1, dtype=jnp.int32))
    bounds = bounds.astype(jnp.int32)
    bnd8 = jnp.broadcast_to(bounds[1:9, None], (8, 16))

    asrc, adst, meta = _prep_kernel(src, dst, bnd8)

    for lvl in range(1, L):
        lvl2 = jnp.full((1, 1), lvl, jnp.int32)
        lvl_arr = jnp.full((16,), lvl, jnp.int32)
        zpart = _seg_kernel(asrc, adst, meta, g, lvl_arr, zinit)
        h, g = _tc_level(zpart, deg3, S, nl2, po2, lvl2, h, g,
                         W_n1, b_n1r, W_n2, b_n2r)

    pos_pad = jnp.concatenate(
        [POs, jnp.zeros((NW * PQ - P,), jnp.int32)])
    hg_pad = _po_kernel(h, pos_pad)
    hg = hg_pad[:P]

    return _tc_final(hg, PO_feat, W_g1r, b_g1r, W_g2, b_g2r,
                     W_o1, b_o1r, W_o2, b_o2r)


# speedup vs baseline: 1.3281x; 1.0409x over previous
import functools

import jax
import jax.numpy as jnp
from jax import lax
from jax.experimental import pallas as pl
from jax.experimental.pallas import tpu as pltpu
from jax.experimental.pallas import tpu_sc as plsc

_SC_PARAMS = pltpu.CompilerParams(needs_layout_passes=False,
                                  use_tc_tiling_on_sc=False)

N = 10000
E = 320000
D = 128
H = 128
HALF = 64
L = 8
P = 1000

NC = 2
NS = 16
NW = NC * NS
CH = E // NW
NSTEP = CH // 16
BLK = 128
NPAD = 10240
RPT = NPAD // NS
DUMP = N


def _leaky(x):
    return jnp.where(x > 0, x, 0.1 * x)


ASZ = CH + 1024


def _prep_body(src_hbm, dst_hbm, bnd_hbm, asrc_hbm, adst_hbm, meta_hbm,
               src_v, dst_v, bnd_v, asrc_v, adst_v, meta_v, sem):
    core = lax.axis_index("c")
    sub = lax.axis_index("s")
    wid = sub * NC + core

    d1 = pltpu.async_copy(src_hbm.at[pl.ds(wid * CH, CH)], src_v, sem)
    d2 = pltpu.async_copy(dst_hbm.at[pl.ds(wid * CH, CH)], dst_v, sem)
    d3 = pltpu.async_copy(bnd_hbm, bnd_v, sem)
    d1.wait(); d2.wait(); d3.wait()

    bnds = [bnd_v[l] for l in range(8)]

    def _lev(d):
        lev = (d >= bnds[0]).astype(jnp.int32)
        for l in range(1, 8):
            lev = lev + (d >= bnds[l]).astype(jnp.int32)
        return lev

    def count_step(i, accs):
        d = dst_v[pl.ds(i * 16, 16)]
        lev = _lev(d)
        return tuple(accs[l - 1] + (lev == l).astype(jnp.int32)
                     for l in range(1, 8))

    accs = lax.fori_loop(0, NSTEP, count_step,
                         tuple(jnp.zeros((16,), jnp.int32)
                               for _ in range(7)))
    cnts = [jnp.sum(a) for a in accs]

    bases = []
    b = jnp.int32(0)
    for l in range(7):
        bases.append(b)
        b = b + ((cnts[l] + (BLK - 1)) // BLK) * BLK

    for l in range(7):
        for k in range(8):
            asrc_v[pl.ds(bases[l] + cnts[l] + 16 * k, 16)] = \
                jnp.zeros((16,), jnp.int32)
            adst_v[pl.ds(bases[l] + cnts[l] + 16 * k, 16)] = \
                jnp.full((16,), DUMP, jnp.int32)

    def compact_step(i, cs):
        d = dst_v[pl.ds(i * 16, 16)]
        s = src_v[pl.ds(i * 16, 16)]
        lev = _lev(d)
        out = []
        for l in range(1, 8):
            m = lev == l
            c = cs[l - 1]
            plsc.store_compressed(asrc_v.at[pl.ds(c, 16)], s, mask=m)
            plsc.store_compressed(adst_v.at[pl.ds(c, 16)], d, mask=m)
            out.append(c + jnp.sum(m.astype(jnp.int32)))
        return tuple(out)

    lax.fori_loop(0, NSTEP, compact_step, tuple(bases))

    for l in range(7):
        meta_v[l] = jnp.broadcast_to(bases[l], (16,))
        meta_v[l + 7] = jnp.broadcast_to(cnts[l], (16,))

    o1 = pltpu.async_copy(asrc_v, asrc_hbm.at[wid], sem)
    o2 = pltpu.async_copy(adst_v, adst_hbm.at[wid], sem)
    o3 = pltpu.async_copy(meta_v, meta_hbm.at[wid], sem)
    o1.wait(); o2.wait(); o3.wait()


_prep_kernel = pl.kernel(
    _prep_body,
    out_type=(jax.ShapeDtypeStruct((NW, ASZ), jnp.int32),
              jax.ShapeDtypeStruct((NW, ASZ), jnp.int32),
              jax.ShapeDtypeStruct((NW, 16, 16), jnp.int32)),
    mesh=plsc.VectorSubcoreMesh(core_axis_name="c", subcore_axis_name="s"),
    scratch_types=[
        pltpu.VMEM((CH,), jnp.int32),
        pltpu.VMEM((CH,), jnp.int32),
        pltpu.VMEM((8, 16), jnp.int32),
        pltpu.VMEM((ASZ,), jnp.int32),
        pltpu.VMEM((ASZ,), jnp.int32),
        pltpu.VMEM((16, 16), jnp.int32),
        pltpu.SemaphoreType.DMA,
    ],
    compiler_params=_SC_PARAMS,
)


def _seg_body(asrc_hbm, adst_hbm, meta_hbm, g_hbm, lvl_hbm, zinit_hbm,
              zout_hbm, meta_v, lvl_v, bsrc_v, bdst_v, cd2_v, rows_v, zsp,
              isem, gsem, ssem):
    core = lax.axis_index("c")
    sub = lax.axis_index("s")
    wid = sub * NC + core

    d1 = pltpu.async_copy(meta_hbm.at[wid], meta_v, isem)
    d2 = pltpu.async_copy(lvl_hbm, lvl_v, isem)
    d3 = pltpu.async_copy(zinit_hbm, zsp.at[pl.ds(sub * RPT, RPT)], isem)
    d4 = pltpu.async_copy(asrc_hbm.at[wid], bsrc_v, isem)
    d5 = pltpu.async_copy(adst_hbm.at[wid], bdst_v, isem)
    d1.wait(); d2.wait(); d3.wait(); d4.wait(); d5.wait()
    plsc.subcore_barrier()

    l = lvl_v[...][0]
    base = pl.multiple_of(meta_v[l - 1][0], BLK)
    cnt = meta_v[l + 6][0]
    nblk = (cnt + (BLK - 1)) // BLK

    def gather(j, p):
        pltpu.async_copy(g_hbm.at[bsrc_v.at[pl.ds(base + j * BLK, BLK)]],
                         rows_v.at[p], gsem)

    def wait_gather(j, p):
        pltpu.make_async_copy(
            g_hbm.at[bsrc_v.at[pl.ds(base + j * BLK, BLK)]],
            rows_v.at[p], gsem).wait()

    def wait_scatter(p):
        pltpu.make_async_copy(rows_v.at[p], zsp.at[cd2_v.at[p]],
                              ssem).wait()

    for k in range(4):
        @pl.when(k < nblk)
        def _(k=k):
            gather(k, k)

    def block(j, _):
        p = lax.rem(j, 8)
        wait_gather(j, p)
        for k in range(BLK // 16):
            cd2_v[p, pl.ds(16 * k, 16)] = \
                bdst_v[pl.ds(base + j * BLK + 16 * k, 16)]
        pltpu.async_copy(rows_v.at[p], zsp.at[cd2_v.at[p]], ssem,
                         add=True)
        jn = j + 4

        @pl.when(jn < nblk)
        def _():
            @pl.when(j >= 4)
            def _():
                wait_scatter(lax.rem(j - 4, 8))
            gather(jn, lax.rem(jn, 8))
        return 0

    lax.fori_loop(0, nblk, block, 0)

    for k in range(8):
        @pl.when(k < jnp.minimum(nblk, 8))
        def _():
            wait_scatter(0)

    plsc.subcore_barrier()
    pltpu.sync_copy(zsp.at[pl.ds(sub * RPT, RPT)],
                    zout_hbm.at[core, pl.ds(sub * RPT, RPT)])


_seg_kernel = pl.kernel(
    _seg_body,
    out_type=jax.ShapeDtypeStruct((NC, NPAD, HALF), jnp.float32),
    mesh=plsc.VectorSubcoreMesh(core_axis_name="c", subcore_axis_name="s"),
    scratch_types=[
        pltpu.VMEM((16, 16), jnp.int32),
        pltpu.VMEM((16,), jnp.int32),
        pltpu.VMEM((ASZ,), jnp.int32),
        pltpu.VMEM((ASZ,), jnp.int32),
        pltpu.VMEM((8, BLK), jnp.int32),
        pltpu.VMEM((8, BLK, HALF), jnp.float32),
        pltpu.VMEM_SHARED((NPAD, HALF), jnp.float32),
        pltpu.SemaphoreType.DMA,
        pltpu.SemaphoreType.DMA,
        pltpu.SemaphoreType.DMA,
    ],
    compiler_params=_SC_PARAMS,
)


DEG_NB = CH // BLK + 1


def _deg_body(dst_hbm, zinit1_hbm, degout_hbm, dst_v, idxr_v, ones_v, dsp,
              sem, ssem):
    core = lax.axis_index("c")
    sub = lax.axis_index("s")
    wid = sub * NC + core

    d1 = pltpu.async_copy(dst_hbm.at[pl.ds(wid * CH, CH)], dst_v, sem)
    d2 = pltpu.async_copy(zinit1_hbm, dsp.at[pl.ds(sub * RPT, RPT)], sem)
    for k in range(BLK // 16):
        ones_v[pl.ds(16 * k, 16)] = jnp.ones((16,), jnp.float32)
    d1.wait(); d2.wait()
    plsc.subcore_barrier()

    nfull = CH // BLK

    def wait_scatter(slot):
        pltpu.make_async_copy(ones_v, dsp.at[idxr_v.at[slot]], ssem).wait()

    def block_scatter(j, _):
        slot = lax.rem(j, 4)

        @pl.when(j >= 4)
        def _():
            wait_scatter(slot)

        for k in range(BLK // 16):
            idxr_v[slot, pl.ds(16 * k, 16)] = \
                dst_v[pl.ds(j * BLK + 16 * k, 16)]
        pltpu.async_copy(ones_v, dsp.at[idxr_v.at[slot]], ssem, add=True)
        return 0

    lax.fori_loop(0, nfull, block_scatter, 0)

    slot = lax.rem(jnp.int32(nfull), 4)
    wait_scatter(slot)
    idxr_v[slot, pl.ds(0, 16)] = dst_v[pl.ds(nfull * BLK, 16)]
    for k in range(1, BLK // 16):
        idxr_v[slot, pl.ds(16 * k, 16)] = jnp.full((16,), DUMP, jnp.int32)
    pltpu.async_copy(ones_v, dsp.at[idxr_v.at[slot]], ssem, add=True)

    for k in range(4):
        wait_scatter(jnp.int32(k))

    plsc.subcore_barrier()
    pltpu.sync_copy(dsp.at[pl.ds(sub * RPT, RPT)],
                    degout_hbm.at[core, pl.ds(sub * RPT, RPT)])


_deg_kernel = pl.kernel(
    _deg_body,
    out_type=jax.ShapeDtypeStruct((NC, NPAD), jnp.float32),
    mesh=plsc.VectorSubcoreMesh(core_axis_name="c", subcore_axis_name="s"),
    scratch_types=[
        pltpu.VMEM((CH,), jnp.int32),
        pltpu.VMEM((4, BLK), jnp.int32),
        pltpu.VMEM((BLK,), jnp.float32),
        pltpu.VMEM_SHARED((NPAD,), jnp.float32),
        pltpu.SemaphoreType.DMA,
        pltpu.SemaphoreType.DMA,
    ],
    compiler_params=_SC_PARAMS,
)


PQ = 32


def _po_body(h_hbm, pos_hbm, hg_hbm, idx_v, rows_v, sem):
    core = lax.axis_index("c")
    sub = lax.axis_index("s")
    wid = sub * NC + core
    pltpu.sync_copy(pos_hbm.at[pl.ds(wid * PQ, PQ)], idx_v)
    pltpu.async_copy(h_hbm.at[idx_v], rows_v, sem).wait()
    pltpu.sync_copy(rows_v, hg_hbm.at[pl.ds(wid * PQ, PQ)])


_po_kernel = pl.kernel(
    _po_body,
    out_type=jax.ShapeDtypeStruct((NW * PQ, H), jnp.float32),
    mesh=plsc.VectorSubcoreMesh(core_axis_name="c", subcore_axis_name="s"),
    scratch_types=[
        pltpu.VMEM((PQ,), jnp.int32),
        pltpu.VMEM((PQ, H), jnp.float32),
        pltpu.SemaphoreType.DMA,
    ],
    compiler_params=_SC_PARAMS,
)


BA = 400
NBA = N // BA


def _init_body(feat_r, delay_r, nl_r, ws1_r, bs1_r, ws2_r, bs2_r,
               wp1_r, bp1_r, wp2_r, bp2_r, wn1_r,
               s_o, h_o, g_o):
    x = feat_r[...]
    s1 = _leaky(jnp.dot(x, ws1_r[...], preferred_element_type=jnp.float32)
                + bs1_r[...])
    s_o[...] = jnp.dot(s1, ws2_r[...], preferred_element_type=jnp.float32) \
        + bs2_r[...]
    d = delay_r[...]
    p1 = _leaky(d * wp1_r[...] + bp1_r[...])
    hp = jnp.dot(p1, wp2_r[...], preferred_element_type=jnp.float32) \
        + bp2_r[...]
    m0 = nl_r[...] == 0
    hblk = jnp.where(m0, hp, 0.0)
    h_o[...] = hblk
    g_o[...] = jnp.dot(hblk, wn1_r[...], preferred_element_type=jnp.float32)


def _tc_init(feat, delay, nl2, W_s1, b_s1, W_s2, b_s2,
             W_pi1, b_pi1, W_pi2, b_pi2, W_n1):
    full = lambda i: (0, 0)
    row = lambda i: (i, 0)
    return pl.pallas_call(
        _init_body,
        grid=(NBA,),
        in_specs=[
            pl.BlockSpec((BA, D), row),
            pl.BlockSpec((BA, 1), row),
            pl.BlockSpec((BA, 1), row),
            pl.BlockSpec((D, HALF), full),
            pl.BlockSpec((1, HALF), full),
            pl.BlockSpec((HALF, H), full),
            pl.BlockSpec((1, H), full),
            pl.BlockSpec((1, HALF), full),
            pl.BlockSpec((1, HALF), full),
            pl.BlockSpec((HALF, H), full),
            pl.BlockSpec((1, H), full),
            pl.BlockSpec((H, HALF), full),
        ],
        out_specs=[
            pl.BlockSpec((BA, H), row),
            pl.BlockSpec((BA, H), row),
            pl.BlockSpec((BA, HALF), row),
        ],
        out_shape=[
            jax.ShapeDtypeStruct((N, H), jnp.float32),
            jax.ShapeDtypeStruct((N, H), jnp.float32),
            jax.ShapeDtypeStruct((N, HALF), jnp.float32),
        ],
    )(feat, delay, nl2, W_s1, b_s1, W_s2, b_s2,
      W_pi1, b_pi1, W_pi2, b_pi2, W_n1)


BD = 1000
NBD = N // BD


def _lvl_body(z_r, dg_r, s_r, nl_r, po_r, lvl_r, h_r, g_r,
              wn1_r, bn1_r, wn2_r, bn2_r, h_o, g_o):
    z = z_r[0] + z_r[1]
    deg = jnp.maximum(dg_r[0] + dg_r[1], 1.0)
    zn = z / deg
    a1 = _leaky(zn + bn1_r[...])
    t = jnp.dot(a1, wn2_r[...], preferred_element_type=jnp.float32) \
        + bn2_r[...] + s_r[...]
    t = jnp.where(po_r[...] != 1, jnp.maximum(t, 0.0), t)
    m = nl_r[...] == lvl_r[...]
    h_o[...] = jnp.where(m, t, h_r[...])
    g_o[...] = jnp.where(
        m, jnp.dot(t, wn1_r[...], preferred_element_type=jnp.float32),
        g_r[...])


def _tc_level(zpart, deg3, S, nl2, po2, lvl2, h, g,
              W_n1, b_n1, W_n2, b_n2):
    full = lambda i: (0, 0)
    row = lambda i: (i, 0)
    return pl.pallas_call(
        _lvl_body,
        grid=(NBD,),
        in_specs=[
            pl.BlockSpec((NC, BD, HALF), lambda i: (0, i, 0)),
            pl.BlockSpec((NC, BD, 1), lambda i: (0, i, 0)),
            pl.BlockSpec((BD, H), row),
            pl.BlockSpec((BD, 1), row),
            pl.BlockSpec((BD, 1), row),
            pl.BlockSpec((1, 1), full),
            pl.BlockSpec((BD, H), row),
            pl.BlockSpec((BD, HALF), row),
            pl.BlockSpec((H, HALF), full),
            pl.BlockSpec((1, HALF), full),
            pl.BlockSpec((HALF, H), full),
            pl.BlockSpec((1, H), full),
        ],
        out_specs=[
            pl.BlockSpec((BD, H), row),
            pl.BlockSpec((BD, HALF), row),
        ],
        out_shape=[
            jax.ShapeDtypeStruct((N, H), jnp.float32),
            jax.ShapeDtypeStruct((N, HALF), jnp.float32),
        ],
    )(zpart, deg3, S, nl2, po2, lvl2, h, g, W_n1, b_n1, W_n2, b_n2)


def _fin_body(hg_r, pf_r, wg1_r, bg1_r, wg2_r, bg2_r,
              wo1_r, bo1_r, wo2_r, bo2_r, out_o):
    q1 = _leaky(pf_r[...] * wg1_r[...] + bg1_r[...])
    hglob = jnp.dot(q1, wg2_r[...], preferred_element_type=jnp.float32) \
        + bg2_r[...]
    u = jnp.dot(hg_r[...], wo1_r[0:H, :],
                preferred_element_type=jnp.float32) \
        + jnp.dot(hglob, wo1_r[H:2 * H, :],
                  preferred_element_type=jnp.float32) + bo1_r[...]
    u = _leaky(u)
    out_o[...] = jnp.dot(u, wo2_r[...], preferred_element_type=jnp.float32) \
        + bo2_r[...]


def _tc_final(hg, PO_feat, W_g1, b_g1, W_g2, b_g2, W_o1, b_o1, W_o2, b_o2):
    return pl.pallas_call(
        _fin_body,
        out_shape=jax.ShapeDtypeStruct((P, 1), jnp.float32),
    )(hg, PO_feat, W_g1, b_g1, W_g2, b_g2, W_o1, b_o1, W_o2, b_o2)


def kernel(feat, delay, edge_index, node_level, is_po, POs, PO_feat,
           W_pi1, b_pi1, W_pi2, b_pi2,
           W_n1, b_n1, W_n2, b_n2,
           W_s1, b_s1, W_s2, b_s2,
           W_g1, b_g1, W_g2, b_g2,
           W_o1, b_o1, W_o2, b_o2):
    f32 = jnp.float32
    nl2 = node_level.reshape(N, 1)
    po2 = is_po.reshape(N, 1)
    b_pi1r = b_pi1.reshape(1, HALF)
    b_pi2r = b_pi2.reshape(1, H)
    b_n1r = b_n1.reshape(1, HALF)
    b_n2r = b_n2.reshape(1, H)
    b_s1r = b_s1.reshape(1, HALF)
    b_s2r = b_s2.reshape(1, H)
    b_g1r = b_g1.reshape(1, HALF)
    b_g2r = b_g2.reshape(1, H)
    b_o1r = b_o1.reshape(1, H)
    b_o2r = b_o2.reshape(1, 1)
    W_g1r = W_g1.reshape(1, HALF)
    W_pi1r = W_pi1.reshape(1, HALF)

    zinit = jnp.zeros((RPT, HALF), f32)
    zinit1 = jnp.zeros((RPT,), f32)

    S, h, g = _tc_init(feat, delay, nl2, W_s1, b_s1r, W_s2, b_s2r,
                       W_pi1r, b_pi1r, W_pi2, b_pi2r, W_n1)

    src = edge_index[0]
    dst = edge_index[1]
    degout = _deg_kernel(dst, zinit1)
    deg3 = degout.reshape(NC, NPAD, 1)

    bounds = jnp.searchsorted(node_level, jnp.arange(L + 1, dtype=jnp.int32))
    bounds = bounds.astype(jnp.int32)
    bnd8 = jnp.broadcast_to(bounds[1:9, None], (8, 16))

    asrc, adst, meta = _prep_kernel(src, dst, bnd8)

    for lvl in range(1, L):
        lvl2 = jnp.full((1, 1), lvl, jnp.int32)
        lvl_arr = jnp.full((16,), lvl, jnp.int32)
        zpart = _seg_kernel(asrc, adst, meta, g, lvl_arr, zinit)
        h, g = _tc_level(zpart, deg3, S, nl2, po2, lvl2, h, g,
                         W_n1, b_n1r, W_n2, b_n2r)

    pos_pad = jnp.concatenate(
        [POs, jnp.zeros((NW * PQ - P,), jnp.int32)])
    hg_pad = _po_kernel(h, pos_pad)
    hg = hg_pad[:P]

    return _tc_final(hg, PO_feat, W_g1r, b_g1r, W_g2, b_g2r,
                     W_o1, b_o1r, W_o2, b_o2r)
